# Initial kernel scaffold; baseline (speedup 1.0000x reference)
import jax
import jax.numpy as jnp
from jax.experimental import pallas as pl


def kernel(x, edge_index):
    raise NotImplementedError("write your pallas kernel here")


# speedup vs baseline: 5.8646x; 5.8646x over previous
import functools

import jax
import jax.numpy as jnp
from jax import lax
from jax.experimental import pallas as pl
from jax.experimental.pallas import tpu as pltpu
from jax.experimental.pallas import tpu_sc as plsc

N = 10000
E = 320000
D = 128
NUM_WL = 3

NC = 2
NS = 16
L = 16
NW = NC * NS

NP = 10240
RPW = NP // NW
UCH = 80
EC = 128
NCHUNK = E // EC
FULL_ROUNDS = NCHUNK // NW
REM = NCHUNK - FULL_ROUNDS * NW
ZR = 128
RPS = NP // NS
ZCOPIES = RPS // ZR

_params = pltpu.CompilerParams(use_tc_tiling_on_sc=False)

_mesh = functools.partial(
    plsc.VectorSubcoreMesh,
    core_axis_name="c",
    subcore_axis_name="s",
    num_cores=NC,
    num_subcores=NS,
)


def _worker_id():
    return lax.axis_index("s") * NC + lax.axis_index("c")


def _fill_2d(ref, nrows, ncols, value):
    vec = jnp.full((L,), value, jnp.float32)

    def row(i, _):
        for cb in range(ncols // L):
            ref[i, pl.ds(cb * L, L)] = vec
        return _

    lax.fori_loop(0, nrows, row, None)


def _scatter_body(h_hbm, src_hbm, dst_hbm, p_hbm, acc_sh, zbuf, sidx,
                  didx, rows, sem):
    cid = lax.axis_index("c")
    sid = lax.axis_index("s")
    wid = _worker_id()

    _fill_2d(zbuf, ZR, D, 0.0)
    for k in range(ZCOPIES):
        pltpu.sync_copy(zbuf, acc_sh.at[pl.ds(sid * RPS + k * ZR, ZR)])
    plsc.subcore_barrier()

    def chunk(c):
        e0 = c * EC
        pltpu.sync_copy(src_hbm.at[pl.ds(e0, EC)], sidx)
        pltpu.sync_copy(dst_hbm.at[pl.ds(e0, EC)], didx)
        pltpu.async_copy(h_hbm.at[sidx], rows, sem).wait()
        pltpu.sync_copy(rows, acc_sh.at[didx], add=True)

    def round_(j, _):
        chunk(wid + j * NW)
        return _

    lax.fori_loop(0, FULL_ROUNDS, round_, None)

    @pl.when(wid < REM)
    def _():
        chunk(FULL_ROUNDS * NW + wid)

    plsc.subcore_barrier()

    for k in range(ZCOPIES):
        r0 = sid * RPS + k * ZR
        pltpu.sync_copy(acc_sh.at[pl.ds(r0, ZR)], zbuf)
        pltpu.sync_copy(zbuf, p_hbm.at[cid, pl.ds(r0, ZR)])


def _make_scatter():
    out_type = jax.ShapeDtypeStruct((NC, NP, D), jnp.float32)
    scratch = [
        pltpu.VMEM_SHARED((NP, D), jnp.float32),
        pltpu.VMEM((ZR, D), jnp.float32),
        pltpu.VMEM((EC,), jnp.int32),
        pltpu.VMEM((EC,), jnp.int32),
        pltpu.VMEM((EC, D), jnp.float32),
        pltpu.SemaphoreType.DMA,
    ]

    def body(h, src, dst, p, acc, zbuf, sidx, didx, rows, sem):
        _scatter_body(h, src, dst, p, acc, zbuf, sidx, didx, rows, sem)

    return pl.kernel(
        body,
        out_type=out_type,
        mesh=_mesh(),
        scratch_types=scratch,
        compiler_params=_params,
    )


def _make_deg():
    out_type = jax.ShapeDtypeStruct((NC, NP, L), jnp.float32)
    scratch = [
        pltpu.VMEM_SHARED((NP, L), jnp.float32),
        pltpu.VMEM((RPS, L), jnp.float32),
        pltpu.VMEM((EC,), jnp.int32),
        pltpu.VMEM((EC, L), jnp.float32),
    ]

    def body(dst_hbm, dp_hbm, dacc_sh, dbuf, didx, ones_b):
        cid = lax.axis_index("c")
        sid = lax.axis_index("s")
        wid = _worker_id()

        _fill_2d(dbuf, RPS, L, 0.0)
        pltpu.sync_copy(dbuf, dacc_sh.at[pl.ds(sid * RPS, RPS)])
        _fill_2d(ones_b, EC, L, 1.0)
        plsc.subcore_barrier()

        def chunk(c):
            pltpu.sync_copy(dst_hbm.at[pl.ds(c * EC, EC)], didx)
            pltpu.sync_copy(ones_b, dacc_sh.at[didx], add=True)

        def round_(j, _):
            chunk(wid + j * NW)
            return _

        lax.fori_loop(0, FULL_ROUNDS, round_, None)

        @pl.when(wid < REM)
        def _():
            chunk(FULL_ROUNDS * NW + wid)

        plsc.subcore_barrier()
        pltpu.sync_copy(dacc_sh.at[pl.ds(sid * RPS, RPS)], dbuf)
        pltpu.sync_copy(dbuf, dp_hbm.at[cid, pl.ds(sid * RPS, RPS)])

    return pl.kernel(
        body,
        out_type=out_type,
        mesh=_mesh(),
        scratch_types=scratch,
        compiler_params=_params,
    )


def _update_rows(hbuf, p0buf, p1buf, rdbuf, nrows):
    def row(i, _):
        vd = rdbuf[i, :]
        for cb in range(D // L):
            s = pl.ds(cb * L, L)
            agg = p0buf[i, s] + p1buf[i, s]
            hbuf[i, s] = 0.5 * (hbuf[i, s] + agg * vd)
        return _

    lax.fori_loop(0, nrows, row, None)


def _make_update_first():
    scratch = [
        pltpu.VMEM((UCH, D), jnp.float32),
        pltpu.VMEM((UCH, D), jnp.float32),
        pltpu.VMEM((UCH, D), jnp.float32),
        pltpu.VMEM((UCH, L), jnp.float32),
        pltpu.VMEM((UCH, L), jnp.float32),
        pltpu.VMEM((UCH, L), jnp.float32),
    ]

    def body(h_hbm, p_hbm, dp_hbm, hout_hbm, rd_hbm, hbuf, p0buf, p1buf,
             d0buf, d1buf, rdbuf):
        wid = _worker_id()
        base = wid * RPW
        one = jnp.full((L,), 1.0, jnp.float32)
        for j in range(RPW // UCH):
            r0 = base + j * UCH
            pltpu.sync_copy(h_hbm.at[pl.ds(r0, UCH)], hbuf)
            pltpu.sync_copy(p_hbm.at[0, pl.ds(r0, UCH)], p0buf)
            pltpu.sync_copy(p_hbm.at[1, pl.ds(r0, UCH)], p1buf)
            pltpu.sync_copy(dp_hbm.at[0, pl.ds(r0, UCH)], d0buf)
            pltpu.sync_copy(dp_hbm.at[1, pl.ds(r0, UCH)], d1buf)

            def drow(i, _):
                d = d0buf[i, :] + d1buf[i, :]
                rdbuf[i, :] = one / jnp.maximum(d, one)
                return _

            lax.fori_loop(0, UCH, drow, None)
            _update_rows(hbuf, p0buf, p1buf, rdbuf, UCH)
            pltpu.sync_copy(hbuf, hout_hbm.at[pl.ds(r0, UCH)])
            pltpu.sync_copy(rdbuf, rd_hbm.at[pl.ds(r0, UCH)])

    return pl.kernel(
        body,
        out_type=(
            jax.ShapeDtypeStruct((NP, D), jnp.float32),
            jax.ShapeDtypeStruct((NP, L), jnp.float32),
        ),
        mesh=_mesh(),
        scratch_types=scratch,
        compiler_params=_params,
    )


def _make_update():
    scratch = [
        pltpu.VMEM((UCH, D), jnp.float32),
        pltpu.VMEM((UCH, D), jnp.float32),
        pltpu.VMEM((UCH, D), jnp.float32),
        pltpu.VMEM((UCH, L), jnp.float32),
    ]

    def body(h_hbm, p_hbm, rd_hbm, hout_hbm, hbuf, p0buf, p1buf, rdbuf):
        wid = _worker_id()
        base = wid * RPW
        for j in range(RPW // UCH):
            r0 = base + j * UCH
            pltpu.sync_copy(h_hbm.at[pl.ds(r0, UCH)], hbuf)
            pltpu.sync_copy(p_hbm.at[0, pl.ds(r0, UCH)], p0buf)
            pltpu.sync_copy(p_hbm.at[1, pl.ds(r0, UCH)], p1buf)
            pltpu.sync_copy(rd_hbm.at[pl.ds(r0, UCH)], rdbuf)
            _update_rows(hbuf, p0buf, p1buf, rdbuf, UCH)
            pltpu.sync_copy(hbuf, hout_hbm.at[pl.ds(r0, UCH)])

    return pl.kernel(
        body,
        out_type=jax.ShapeDtypeStruct((NP, D), jnp.float32),
        mesh=_mesh(),
        scratch_types=scratch,
        compiler_params=_params,
    )


_deg = _make_deg()
_scatter = _make_scatter()
_update_first = _make_update_first()
_update = _make_update()


@jax.jit
def kernel(x, edge_index):
    src = edge_index[0]
    dst = edge_index[1]
    xp = jnp.zeros((NP, D), jnp.float32).at[:N].set(x[:, :D])

    dp = _deg(dst)
    p = _scatter(xp, src, dst)
    h1, rdeg = _update_first(xp, p, dp)
    p2 = _scatter(h1, src, dst)
    h2 = _update(h1, p2, rdeg)
    p3 = _scatter(h2, src, dst)
    h3 = _update(h2, p3, rdeg)

    return jnp.concatenate([xp[:N], h1[:N], h2[:N], h3[:N]], axis=1)


# speedup vs baseline: 9.2505x; 1.5774x over previous
import functools

import jax
import jax.numpy as jnp
from jax import lax
from jax.experimental import pallas as pl
from jax.experimental.pallas import tpu as pltpu
from jax.experimental.pallas import tpu_sc as plsc

N = 10000
E = 320000
D = 128
NUM_WL = 3

NC = 2
NS = 16
L = 16
NW = NC * NS

NP = 10240
RPW = NP // NW
UCH = 80
EC = 128
NCHUNK = E // EC
FULL_ROUNDS = NCHUNK // NW
REM = NCHUNK - FULL_ROUNDS * NW
ZR = 128
RPS = NP // NS
ZCOPIES = RPS // ZR

_params = pltpu.CompilerParams(use_tc_tiling_on_sc=False)

_mesh = functools.partial(
    plsc.VectorSubcoreMesh,
    core_axis_name="c",
    subcore_axis_name="s",
    num_cores=NC,
    num_subcores=NS,
)


def _worker_id():
    return lax.axis_index("s") * NC + lax.axis_index("c")


def _fill_2d(ref, nrows, ncols, value):
    vec = jnp.full((L,), value, jnp.float32)

    def row(i, _):
        for cb in range(ncols // L):
            ref[i, pl.ds(cb * L, L)] = vec
        return _

    lax.fori_loop(0, nrows, row, None)


EPW = E // NW
FCH = -(-EPW // EC)
PADE = FCH * EC - EPW
EPWP = FCH * EC
NB = 2


def _scatter_body(h_hbm, src_hbm, dst_hbm, z_hbm, p_hbm, acc_sh, sidx_all,
                  didx, rows, sem_i, sem_g):
    cid = lax.axis_index("c")
    sid = lax.axis_index("s")
    wid = _worker_id()
    ebase = wid * EPW

    pltpu.sync_copy(z_hbm, acc_sh.at[pl.ds(sid * RPS, RPS)])
    plsc.subcore_barrier()

    pltpu.sync_copy(src_hbm.at[pl.ds(ebase, EPW)],
                    sidx_all.at[pl.ds(0, EPW)])
    lanes = lax.iota(jnp.int32, L)
    for k in range(PADE // L):
        sidx_all[pl.ds(EPW + k * L, L)] = lanes + (k * L)

    def start_didx(j, b):
        pltpu.async_copy(dst_hbm.at[pl.ds(ebase + j * EC, EC)], didx[b],
                         sem_i)

    def start_gather(j, b):
        pltpu.async_copy(h_hbm.at[sidx_all.at[pl.ds(j * EC, EC)]], rows[b],
                         sem_g)

    def wait_didx(b):
        pltpu.make_async_copy(dst_hbm.at[pl.ds(0, EC)], didx[b],
                              sem_i).wait()

    def wait_gather(b):
        pltpu.make_async_copy(h_hbm.at[sidx_all.at[pl.ds(0, EC)]], rows[b],
                              sem_g).wait()

    def scatter_sync(b):
        pltpu.sync_copy(rows[b], acc_sh.at[didx[b]], add=True)

    def load_didx(j, b):
        @pl.when(j < FCH - 1)
        def _():
            start_didx(j, b)

        @pl.when(j == FCH - 1)
        def _():
            pltpu.async_copy(
                dst_hbm.at[pl.ds(ebase + j * EC, EC - PADE)],
                didx[b].at[pl.ds(0, EC - PADE)], sem_i)

    def wait_didx_tail(j, b):
        @pl.when(j < FCH - 1)
        def _():
            wait_didx(b)

        @pl.when(j == FCH - 1)
        def _():
            pltpu.make_async_copy(dst_hbm.at[pl.ds(0, EC - PADE)],
                                  didx[b].at[pl.ds(0, EC - PADE)],
                                  sem_i).wait()
            for k in range(PADE // L):
                didx[b][pl.ds(EC - PADE + k * L, L)] = (
                    lanes + (N + 64 + k * L))

    load_didx(0, 0)
    start_gather(0, 0)

    def pair(g, _):
        for b in range(NB):
            j = g * NB + b

            @pl.when(j < FCH)
            def _():
                wait_didx_tail(j, b)
                wait_gather(b)

            @pl.when(j + 1 < FCH)
            def _():
                load_didx(j + 1, 1 - b)
                start_gather(j + 1, 1 - b)

            @pl.when(j < FCH)
            def _():
                scatter_sync(b)
        return _

    lax.fori_loop(0, (FCH + 1) // NB, pair, None)

    plsc.subcore_barrier()

    r0 = sid * RPS
    pltpu.sync_copy(acc_sh.at[pl.ds(r0, RPS)], p_hbm.at[cid, pl.ds(r0, RPS)])


def _make_scatter():
    out_type = jax.ShapeDtypeStruct((NC, NP, D), jnp.float32)
    scratch = [
        pltpu.VMEM_SHARED((NP, D), jnp.float32),
        pltpu.VMEM((EPWP,), jnp.int32),
        [pltpu.VMEM((EC,), jnp.int32) for _ in range(NB)],
        [pltpu.VMEM((EC, D), jnp.float32) for _ in range(NB)],
        pltpu.SemaphoreType.DMA,
        pltpu.SemaphoreType.DMA,
    ]

    def body(h, src, dst, z, p, acc, sidx_all, didx, rows, sem_i,
             sem_g):
        _scatter_body(h, src, dst, z, p, acc, sidx_all, didx, rows,
                      sem_i, sem_g)

    return pl.kernel(
        body,
        out_type=out_type,
        mesh=_mesh(),
        scratch_types=scratch,
        compiler_params=_params,
    )


def _make_deg():
    out_type = jax.ShapeDtypeStruct((NC, NP, L), jnp.float32)
    scratch = [
        pltpu.VMEM_SHARED((NP, L), jnp.float32),
        pltpu.VMEM((RPS, L), jnp.float32),
        pltpu.VMEM((EC,), jnp.int32),
        pltpu.VMEM((EC, L), jnp.float32),
    ]

    def body(dst_hbm, dp_hbm, dacc_sh, dbuf, didx, ones_b):
        cid = lax.axis_index("c")
        sid = lax.axis_index("s")
        wid = _worker_id()

        _fill_2d(dbuf, RPS, L, 0.0)
        pltpu.sync_copy(dbuf, dacc_sh.at[pl.ds(sid * RPS, RPS)])
        _fill_2d(ones_b, EC, L, 1.0)
        plsc.subcore_barrier()

        def chunk(c):
            pltpu.sync_copy(dst_hbm.at[pl.ds(c * EC, EC)], didx)
            pltpu.sync_copy(ones_b, dacc_sh.at[didx], add=True)

        def round_(j, _):
            chunk(wid + j * NW)
            return _

        lax.fori_loop(0, FULL_ROUNDS, round_, None)

        @pl.when(wid < REM)
        def _():
            chunk(FULL_ROUNDS * NW + wid)

        plsc.subcore_barrier()
        pltpu.sync_copy(dacc_sh.at[pl.ds(sid * RPS, RPS)], dbuf)
        pltpu.sync_copy(dbuf, dp_hbm.at[cid, pl.ds(sid * RPS, RPS)])

    return pl.kernel(
        body,
        out_type=out_type,
        mesh=_mesh(),
        scratch_types=scratch,
        compiler_params=_params,
    )


def _update_rows(hbuf, p0buf, p1buf, rdbuf, nrows):
    def row(i, _):
        vd = rdbuf[i, :]
        for cb in range(D // L):
            s = pl.ds(cb * L, L)
            agg = p0buf[i, s] + p1buf[i, s]
            hbuf[i, s] = 0.5 * (hbuf[i, s] + agg * vd)
        return _

    lax.fori_loop(0, nrows, row, None)


def _make_update_first():
    scratch = [
        pltpu.VMEM((UCH, D), jnp.float32),
        pltpu.VMEM((UCH, D), jnp.float32),
        pltpu.VMEM((UCH, D), jnp.float32),
        pltpu.VMEM((UCH, L), jnp.float32),
        pltpu.VMEM((UCH, L), jnp.float32),
        pltpu.VMEM((UCH, L), jnp.float32),
    ]

    def body(h_hbm, p_hbm, dp_hbm, hout_hbm, rd_hbm, hbuf, p0buf, p1buf,
             d0buf, d1buf, rdbuf):
        wid = _worker_id()
        base = wid * RPW
        one = jnp.full((L,), 1.0, jnp.float32)
        for j in range(RPW // UCH):
            r0 = base + j * UCH
            pltpu.sync_copy(h_hbm.at[pl.ds(r0, UCH)], hbuf)
            pltpu.sync_copy(p_hbm.at[0, pl.ds(r0, UCH)], p0buf)
            pltpu.sync_copy(p_hbm.at[1, pl.ds(r0, UCH)], p1buf)
            pltpu.sync_copy(dp_hbm.at[0, pl.ds(r0, UCH)], d0buf)
            pltpu.sync_copy(dp_hbm.at[1, pl.ds(r0, UCH)], d1buf)

            def drow(i, _):
                d = d0buf[i, :] + d1buf[i, :]
                rdbuf[i, :] = one / jnp.maximum(d, one)
                return _

            lax.fori_loop(0, UCH, drow, None)
            _update_rows(hbuf, p0buf, p1buf, rdbuf, UCH)
            pltpu.sync_copy(hbuf, hout_hbm.at[pl.ds(r0, UCH)])
            pltpu.sync_copy(rdbuf, rd_hbm.at[pl.ds(r0, UCH)])

    return pl.kernel(
        body,
        out_type=(
            jax.ShapeDtypeStruct((NP, D), jnp.float32),
            jax.ShapeDtypeStruct((NP, L), jnp.float32),
        ),
        mesh=_mesh(),
        scratch_types=scratch,
        compiler_params=_params,
    )


def _make_update():
    scratch = [
        pltpu.VMEM((UCH, D), jnp.float32),
        pltpu.VMEM((UCH, D), jnp.float32),
        pltpu.VMEM((UCH, D), jnp.float32),
        pltpu.VMEM((UCH, L), jnp.float32),
    ]

    def body(h_hbm, p_hbm, rd_hbm, hout_hbm, hbuf, p0buf, p1buf, rdbuf):
        wid = _worker_id()
        base = wid * RPW
        for j in range(RPW // UCH):
            r0 = base + j * UCH
            pltpu.sync_copy(h_hbm.at[pl.ds(r0, UCH)], hbuf)
            pltpu.sync_copy(p_hbm.at[0, pl.ds(r0, UCH)], p0buf)
            pltpu.sync_copy(p_hbm.at[1, pl.ds(r0, UCH)], p1buf)
            pltpu.sync_copy(rd_hbm.at[pl.ds(r0, UCH)], rdbuf)
            _update_rows(hbuf, p0buf, p1buf, rdbuf, UCH)
            pltpu.sync_copy(hbuf, hout_hbm.at[pl.ds(r0, UCH)])

    return pl.kernel(
        body,
        out_type=jax.ShapeDtypeStruct((NP, D), jnp.float32),
        mesh=_mesh(),
        scratch_types=scratch,
        compiler_params=_params,
    )


_deg = _make_deg()
_scatter = _make_scatter()
_update_first = _make_update_first()
_update = _make_update()


@jax.jit
def kernel(x, edge_index):
    src = edge_index[0]
    dst = edge_index[1]
    xp = jnp.zeros((NP, D), jnp.float32).at[:N].set(x[:, :D])

    zblk = jnp.zeros((RPS, D), jnp.float32)
    dp = _deg(dst)
    p = _scatter(xp, src, dst, zblk)
    h1, rdeg = _update_first(xp, p, dp)
    p2 = _scatter(h1, src, dst, zblk)
    h2 = _update(h1, p2, rdeg)
    p3 = _scatter(h2, src, dst, zblk)
    h3 = _update(h2, p3, rdeg)

    return jnp.concatenate([xp[:N], h1[:N], h2[:N], h3[:N]], axis=1)


# speedup vs baseline: 9.2618x; 1.0012x over previous
import functools

import jax
import jax.numpy as jnp
from jax import lax
from jax.experimental import pallas as pl
from jax.experimental.pallas import tpu as pltpu
from jax.experimental.pallas import tpu_sc as plsc

N = 10000
E = 320000
D = 128
NUM_WL = 3

NC = 2
NS = 16
L = 16
NW = NC * NS

NP = 10240
RPW = NP // NW
UCH = 80
EC = 128
NCHUNK = E // EC
FULL_ROUNDS = NCHUNK // NW
REM = NCHUNK - FULL_ROUNDS * NW
ZR = 128
RPS = NP // NS
ZCOPIES = RPS // ZR

_params = pltpu.CompilerParams(use_tc_tiling_on_sc=False)

_mesh = functools.partial(
    plsc.VectorSubcoreMesh,
    core_axis_name="c",
    subcore_axis_name="s",
    num_cores=NC,
    num_subcores=NS,
)


def _worker_id():
    return lax.axis_index("s") * NC + lax.axis_index("c")


def _fill_2d(ref, nrows, ncols, value):
    vec = jnp.full((L,), value, jnp.float32)

    def row(i, _):
        for cb in range(ncols // L):
            ref[i, pl.ds(cb * L, L)] = vec
        return _

    lax.fori_loop(0, nrows, row, None)


EPW = E // NW
FCH = -(-EPW // EC)
PADE = FCH * EC - EPW
EPWP = FCH * EC
NB = 2


def _scatter_body(h_hbm, src_hbm, dst_hbm, z_hbm, p_hbm, acc_sh, sidx_all,
                  didx, rows, sem_i, sem_g, sem_s):
    cid = lax.axis_index("c")
    sid = lax.axis_index("s")
    wid = _worker_id()
    ebase = wid * EPW

    pltpu.sync_copy(z_hbm, acc_sh.at[pl.ds(sid * RPS, RPS)])
    plsc.subcore_barrier()

    pltpu.sync_copy(src_hbm.at[pl.ds(ebase, EPW)],
                    sidx_all.at[pl.ds(0, EPW)])
    lanes = lax.iota(jnp.int32, L)
    for k in range(PADE // L):
        sidx_all[pl.ds(EPW + k * L, L)] = lanes + (k * L)

    def start_didx(j, b):
        pltpu.async_copy(dst_hbm.at[pl.ds(ebase + j * EC, EC)], didx[b],
                         sem_i)

    def start_gather(j, b):
        pltpu.async_copy(h_hbm.at[sidx_all.at[pl.ds(j * EC, EC)]], rows[b],
                         sem_g)

    def wait_didx(b):
        pltpu.make_async_copy(dst_hbm.at[pl.ds(0, EC)], didx[b],
                              sem_i).wait()

    def wait_gather(b):
        pltpu.make_async_copy(h_hbm.at[sidx_all.at[pl.ds(0, EC)]], rows[b],
                              sem_g).wait()

    def start_scatter(b):
        pltpu.async_copy(rows[b], acc_sh.at[didx[b]], sem_s, add=True)

    def wait_scatter(b):
        pltpu.make_async_copy(rows[b], acc_sh.at[didx[b]], sem_s).wait()

    def load_didx(j, b):
        @pl.when(j < FCH - 1)
        def _():
            start_didx(j, b)

        @pl.when(j == FCH - 1)
        def _():
            pltpu.async_copy(
                dst_hbm.at[pl.ds(ebase + j * EC, EC - PADE)],
                didx[b].at[pl.ds(0, EC - PADE)], sem_i)

    def wait_didx_tail(j, b):
        @pl.when(j < FCH - 1)
        def _():
            wait_didx(b)

        @pl.when(j == FCH - 1)
        def _():
            pltpu.make_async_copy(dst_hbm.at[pl.ds(0, EC - PADE)],
                                  didx[b].at[pl.ds(0, EC - PADE)],
                                  sem_i).wait()
            for k in range(PADE // L):
                didx[b][pl.ds(EC - PADE + k * L, L)] = (
                    lanes + (N + 64 + k * L))

    load_didx(0, 0)
    start_gather(0, 0)

    def pair(g, _):
        for b in range(NB):
            j = g * NB + b

            @pl.when(j < FCH)
            def _():
                wait_didx_tail(j, b)
                wait_gather(b)

            @pl.when(jnp.logical_and(j >= 1, j <= FCH))
            def _():
                wait_scatter(1 - b)

            @pl.when(j < FCH)
            def _():
                start_scatter(b)

            @pl.when(j + 1 < FCH)
            def _():
                load_didx(j + 1, 1 - b)
                start_gather(j + 1, 1 - b)
        return _

    lax.fori_loop(0, (FCH + 1) // NB, pair, None)

    plsc.subcore_barrier()

    r0 = sid * RPS
    pltpu.sync_copy(acc_sh.at[pl.ds(r0, RPS)], p_hbm.at[cid, pl.ds(r0, RPS)])


def _make_scatter():
    out_type = jax.ShapeDtypeStruct((NC, NP, D), jnp.float32)
    scratch = [
        pltpu.VMEM_SHARED((NP, D), jnp.float32),
        pltpu.VMEM((EPWP,), jnp.int32),
        [pltpu.VMEM((EC,), jnp.int32) for _ in range(NB)],
        [pltpu.VMEM((EC, D), jnp.float32) for _ in range(NB)],
        pltpu.SemaphoreType.DMA,
        pltpu.SemaphoreType.DMA,
        pltpu.SemaphoreType.DMA,
    ]

    def body(h, src, dst, z, p, acc, sidx_all, didx, rows, sem_i,
             sem_g, sem_s):
        _scatter_body(h, src, dst, z, p, acc, sidx_all, didx, rows,
                      sem_i, sem_g, sem_s)

    return pl.kernel(
        body,
        out_type=out_type,
        mesh=_mesh(),
        scratch_types=scratch,
        compiler_params=_params,
    )


def _make_deg():
    out_type = jax.ShapeDtypeStruct((NC, NP, L), jnp.float32)
    scratch = [
        pltpu.VMEM_SHARED((NP, L), jnp.float32),
        pltpu.VMEM((RPS, L), jnp.float32),
        pltpu.VMEM((EC,), jnp.int32),
        pltpu.VMEM((EC, L), jnp.float32),
    ]

    def body(dst_hbm, dp_hbm, dacc_sh, dbuf, didx, ones_b):
        cid = lax.axis_index("c")
        sid = lax.axis_index("s")
        wid = _worker_id()

        _fill_2d(dbuf, RPS, L, 0.0)
        pltpu.sync_copy(dbuf, dacc_sh.at[pl.ds(sid * RPS, RPS)])
        _fill_2d(ones_b, EC, L, 1.0)
        plsc.subcore_barrier()

        def chunk(c):
            pltpu.sync_copy(dst_hbm.at[pl.ds(c * EC, EC)], didx)
            pltpu.sync_copy(ones_b, dacc_sh.at[didx], add=True)

        def round_(j, _):
            chunk(wid + j * NW)
            return _

        lax.fori_loop(0, FULL_ROUNDS, round_, None)

        @pl.when(wid < REM)
        def _():
            chunk(FULL_ROUNDS * NW + wid)

        plsc.subcore_barrier()
        pltpu.sync_copy(dacc_sh.at[pl.ds(sid * RPS, RPS)], dbuf)
        pltpu.sync_copy(dbuf, dp_hbm.at[cid, pl.ds(sid * RPS, RPS)])

    return pl.kernel(
        body,
        out_type=out_type,
        mesh=_mesh(),
        scratch_types=scratch,
        compiler_params=_params,
    )


def _update_rows(hbuf, p0buf, p1buf, rdbuf, nrows):
    def row(i, _):
        vd = rdbuf[i, :]
        for cb in range(D // L):
            s = pl.ds(cb * L, L)
            agg = p0buf[i, s] + p1buf[i, s]
            hbuf[i, s] = 0.5 * (hbuf[i, s] + agg * vd)
        return _

    lax.fori_loop(0, nrows, row, None)


def _make_update_first():
    scratch = [
        pltpu.VMEM((UCH, D), jnp.float32),
        pltpu.VMEM((UCH, D), jnp.float32),
        pltpu.VMEM((UCH, D), jnp.float32),
        pltpu.VMEM((UCH, L), jnp.float32),
        pltpu.VMEM((UCH, L), jnp.float32),
        pltpu.VMEM((UCH, L), jnp.float32),
    ]

    def body(h_hbm, p_hbm, dp_hbm, hout_hbm, rd_hbm, hbuf, p0buf, p1buf,
             d0buf, d1buf, rdbuf):
        wid = _worker_id()
        base = wid * RPW
        one = jnp.full((L,), 1.0, jnp.float32)
        for j in range(RPW // UCH):
            r0 = base + j * UCH
            pltpu.sync_copy(h_hbm.at[pl.ds(r0, UCH)], hbuf)
            pltpu.sync_copy(p_hbm.at[0, pl.ds(r0, UCH)], p0buf)
            pltpu.sync_copy(p_hbm.at[1, pl.ds(r0, UCH)], p1buf)
            pltpu.sync_copy(dp_hbm.at[0, pl.ds(r0, UCH)], d0buf)
            pltpu.sync_copy(dp_hbm.at[1, pl.ds(r0, UCH)], d1buf)

            def drow(i, _):
                d = d0buf[i, :] + d1buf[i, :]
                rdbuf[i, :] = one / jnp.maximum(d, one)
                return _

            lax.fori_loop(0, UCH, drow, None)
            _update_rows(hbuf, p0buf, p1buf, rdbuf, UCH)
            pltpu.sync_copy(hbuf, hout_hbm.at[pl.ds(r0, UCH)])
            pltpu.sync_copy(rdbuf, rd_hbm.at[pl.ds(r0, UCH)])

    return pl.kernel(
        body,
        out_type=(
            jax.ShapeDtypeStruct((NP, D), jnp.float32),
            jax.ShapeDtypeStruct((NP, L), jnp.float32),
        ),
        mesh=_mesh(),
        scratch_types=scratch,
        compiler_params=_params,
    )


def _make_update():
    scratch = [
        pltpu.VMEM((UCH, D), jnp.float32),
        pltpu.VMEM((UCH, D), jnp.float32),
        pltpu.VMEM((UCH, D), jnp.float32),
        pltpu.VMEM((UCH, L), jnp.float32),
    ]

    def body(h_hbm, p_hbm, rd_hbm, hout_hbm, hbuf, p0buf, p1buf, rdbuf):
        wid = _worker_id()
        base = wid * RPW
        for j in range(RPW // UCH):
            r0 = base + j * UCH
            pltpu.sync_copy(h_hbm.at[pl.ds(r0, UCH)], hbuf)
            pltpu.sync_copy(p_hbm.at[0, pl.ds(r0, UCH)], p0buf)
            pltpu.sync_copy(p_hbm.at[1, pl.ds(r0, UCH)], p1buf)
            pltpu.sync_copy(rd_hbm.at[pl.ds(r0, UCH)], rdbuf)
            _update_rows(hbuf, p0buf, p1buf, rdbuf, UCH)
            pltpu.sync_copy(hbuf, hout_hbm.at[pl.ds(r0, UCH)])

    return pl.kernel(
        body,
        out_type=jax.ShapeDtypeStruct((NP, D), jnp.float32),
        mesh=_mesh(),
        scratch_types=scratch,
        compiler_params=_params,
    )


_deg = _make_deg()
_scatter = _make_scatter()
_update_first = _make_update_first()
_update = _make_update()


@jax.jit
def kernel(x, edge_index):
    src = edge_index[0]
    dst = edge_index[1]
    xp = jnp.zeros((NP, D), jnp.float32).at[:N].set(x[:, :D])

    zblk = jnp.zeros((RPS, D), jnp.float32)
    dp = _deg(dst)
    p = _scatter(xp, src, dst, zblk)
    h1, rdeg = _update_first(xp, p, dp)
    p2 = _scatter(h1, src, dst, zblk)
    h2 = _update(h1, p2, rdeg)
    p3 = _scatter(h2, src, dst, zblk)
    h3 = _update(h2, p3, rdeg)

    return jnp.concatenate([xp[:N], h1[:N], h2[:N], h3[:N]], axis=1)


# speedup vs baseline: 9.8550x; 1.0641x over previous
import functools

import jax
import jax.numpy as jnp
from jax import lax
from jax.experimental import pallas as pl
from jax.experimental.pallas import tpu as pltpu
from jax.experimental.pallas import tpu_sc as plsc

N = 10000
E = 320000
D = 128
NUM_WL = 3

NC = 2
NS = 16
L = 16
NW = NC * NS

NP = 10240
RPW = NP // NW
UCH = 80
EC = 128
NCHUNK = E // EC
FULL_ROUNDS = NCHUNK // NW
REM = NCHUNK - FULL_ROUNDS * NW
ZR = 128
RPS = NP // NS
ZCOPIES = RPS // ZR

_params = pltpu.CompilerParams(use_tc_tiling_on_sc=False)

_mesh = functools.partial(
    plsc.VectorSubcoreMesh,
    core_axis_name="c",
    subcore_axis_name="s",
    num_cores=NC,
    num_subcores=NS,
)


def _worker_id():
    return lax.axis_index("s") * NC + lax.axis_index("c")


def _fill_2d(ref, nrows, ncols, value):
    vec = jnp.full((L,), value, jnp.float32)

    def row(i, _):
        for cb in range(ncols // L):
            ref[i, pl.ds(cb * L, L)] = vec
        return _

    lax.fori_loop(0, nrows, row, None)


EPW = E // NW
FCH = -(-EPW // EC)
PADE = FCH * EC - EPW
EPWP = FCH * EC
NB = 2


def _scatter_body(h_hbm, src_hbm, dst_hbm, z_hbm, p_hbm, acc_sh, sidx_all,
                  didx, rows, sem_i, sem_g, sem_s):
    cid = lax.axis_index("c")
    sid = lax.axis_index("s")
    wid = _worker_id()
    ebase = wid * EPW

    pltpu.sync_copy(z_hbm, acc_sh.at[pl.ds(sid * RPS, RPS)])
    plsc.subcore_barrier()

    pltpu.sync_copy(src_hbm.at[pl.ds(ebase, EPW)],
                    sidx_all.at[pl.ds(0, EPW)])
    lanes = lax.iota(jnp.int32, L)
    for k in range(PADE // L):
        sidx_all[pl.ds(EPW + k * L, L)] = lanes + (k * L)

    def start_didx(j, b):
        pltpu.async_copy(dst_hbm.at[pl.ds(ebase + j * EC, EC)], didx[b],
                         sem_i)

    def start_gather(j, b):
        pltpu.async_copy(h_hbm.at[sidx_all.at[pl.ds(j * EC, EC)]], rows[b],
                         sem_g)

    def wait_didx(b):
        pltpu.make_async_copy(dst_hbm.at[pl.ds(0, EC)], didx[b],
                              sem_i).wait()

    def wait_gather(b):
        pltpu.make_async_copy(h_hbm.at[sidx_all.at[pl.ds(0, EC)]], rows[b],
                              sem_g).wait()

    def start_scatter(b):
        pltpu.async_copy(rows[b], acc_sh.at[didx[b]], sem_s, add=True)

    def wait_scatter(b):
        pltpu.make_async_copy(rows[b], acc_sh.at[didx[b]], sem_s).wait()

    def load_didx(j, b):
        @pl.when(j < FCH - 1)
        def _():
            start_didx(j, b)

        @pl.when(j == FCH - 1)
        def _():
            pltpu.async_copy(
                dst_hbm.at[pl.ds(ebase + j * EC, EC - PADE)],
                didx[b].at[pl.ds(0, EC - PADE)], sem_i)

    def wait_didx_tail(j, b):
        @pl.when(j < FCH - 1)
        def _():
            wait_didx(b)

        @pl.when(j == FCH - 1)
        def _():
            pltpu.make_async_copy(dst_hbm.at[pl.ds(0, EC - PADE)],
                                  didx[b].at[pl.ds(0, EC - PADE)],
                                  sem_i).wait()
            for k in range(PADE // L):
                didx[b][pl.ds(EC - PADE + k * L, L)] = (
                    lanes + (N + 64 + k * L))

    load_didx(0, 0)
    start_gather(0, 0)

    def pair(g, _):
        for b in range(NB):
            j = g * NB + b

            @pl.when(j < FCH)
            def _():
                wait_didx_tail(j, b)
                wait_gather(b)

            @pl.when(jnp.logical_and(j >= 1, j <= FCH))
            def _():
                wait_scatter(1 - b)

            @pl.when(j < FCH)
            def _():
                start_scatter(b)

            @pl.when(j + 1 < FCH)
            def _():
                load_didx(j + 1, 1 - b)
                start_gather(j + 1, 1 - b)
        return _

    lax.fori_loop(0, (FCH + 1) // NB, pair, None)

    plsc.subcore_barrier()

    r0 = sid * RPS
    pltpu.sync_copy(acc_sh.at[pl.ds(r0, RPS)], p_hbm.at[cid, pl.ds(r0, RPS)])


def _make_scatter():
    out_type = jax.ShapeDtypeStruct((NC, NP, D), jnp.float32)
    scratch = [
        pltpu.VMEM_SHARED((NP, D), jnp.float32),
        pltpu.VMEM((EPWP,), jnp.int32),
        [pltpu.VMEM((EC,), jnp.int32) for _ in range(NB)],
        [pltpu.VMEM((EC, D), jnp.float32) for _ in range(NB)],
        pltpu.SemaphoreType.DMA,
        pltpu.SemaphoreType.DMA,
        pltpu.SemaphoreType.DMA,
    ]

    def body(h, src, dst, z, p, acc, sidx_all, didx, rows, sem_i,
             sem_g, sem_s):
        _scatter_body(h, src, dst, z, p, acc, sidx_all, didx, rows,
                      sem_i, sem_g, sem_s)

    return pl.kernel(
        body,
        out_type=out_type,
        mesh=_mesh(),
        scratch_types=scratch,
        compiler_params=_params,
    )


def _make_deg():
    out_type = jax.ShapeDtypeStruct((NC, NP, L), jnp.float32)
    scratch = [
        pltpu.VMEM_SHARED((NP, L), jnp.float32),
        pltpu.VMEM((RPS, L), jnp.float32),
        pltpu.VMEM((EC,), jnp.int32),
        pltpu.VMEM((EC, L), jnp.float32),
    ]

    def body(dst_hbm, dp_hbm, dacc_sh, dbuf, didx, ones_b):
        cid = lax.axis_index("c")
        sid = lax.axis_index("s")
        wid = _worker_id()

        _fill_2d(dbuf, RPS, L, 0.0)
        pltpu.sync_copy(dbuf, dacc_sh.at[pl.ds(sid * RPS, RPS)])
        _fill_2d(ones_b, EC, L, 1.0)
        plsc.subcore_barrier()

        def chunk(c):
            pltpu.sync_copy(dst_hbm.at[pl.ds(c * EC, EC)], didx)
            pltpu.sync_copy(ones_b, dacc_sh.at[didx], add=True)

        def round_(j, _):
            chunk(wid + j * NW)
            return _

        lax.fori_loop(0, FULL_ROUNDS, round_, None)

        @pl.when(wid < REM)
        def _():
            chunk(FULL_ROUNDS * NW + wid)

        plsc.subcore_barrier()
        pltpu.sync_copy(dacc_sh.at[pl.ds(sid * RPS, RPS)], dbuf)
        pltpu.sync_copy(dbuf, dp_hbm.at[cid, pl.ds(sid * RPS, RPS)])

    return pl.kernel(
        body,
        out_type=out_type,
        mesh=_mesh(),
        scratch_types=scratch,
        compiler_params=_params,
    )


def _update_rows(hbuf, p0buf, p1buf, rdbuf, nrows):
    def row(i, _):
        vd = rdbuf[i, :]
        for cb in range(D // L):
            s = pl.ds(cb * L, L)
            agg = p0buf[i, s] + p1buf[i, s]
            hbuf[i, s] = 0.5 * (hbuf[i, s] + agg * vd)
        return _

    lax.fori_loop(0, nrows, row, None)


def _make_update_first():
    scratch = [
        [pltpu.VMEM((UCH, D), jnp.float32) for _ in range(2)],
        [pltpu.VMEM((UCH, D), jnp.float32) for _ in range(2)],
        [pltpu.VMEM((UCH, D), jnp.float32) for _ in range(2)],
        [pltpu.VMEM((UCH, L), jnp.float32) for _ in range(2)],
        [pltpu.VMEM((UCH, L), jnp.float32) for _ in range(2)],
        pltpu.VMEM((UCH, L), jnp.float32),
        pltpu.SemaphoreType.DMA,
        pltpu.SemaphoreType.DMA,
    ]

    def body(h_hbm, p_hbm, dp_hbm, hout_hbm, rd_hbm, hbuf, p0buf, p1buf,
             d0buf, d1buf, rdbuf, sem_ld, sem_st):
        wid = _worker_id()
        base = wid * RPW
        one = jnp.full((L,), 1.0, jnp.float32)
        NJ = RPW // UCH

        def load_chunk(j, b):
            r0 = base + j * UCH
            pltpu.async_copy(h_hbm.at[pl.ds(r0, UCH)], hbuf[b], sem_ld)
            pltpu.async_copy(p_hbm.at[0, pl.ds(r0, UCH)], p0buf[b], sem_ld)
            pltpu.async_copy(p_hbm.at[1, pl.ds(r0, UCH)], p1buf[b], sem_ld)
            pltpu.async_copy(dp_hbm.at[0, pl.ds(r0, UCH)], d0buf[b],
                             sem_ld)
            pltpu.async_copy(dp_hbm.at[1, pl.ds(r0, UCH)], d1buf[b],
                             sem_ld)

        def wait_chunk(b):
            pltpu.make_async_copy(h_hbm.at[pl.ds(0, UCH)], hbuf[b],
                                  sem_ld).wait()
            pltpu.make_async_copy(h_hbm.at[pl.ds(0, UCH)], p0buf[b],
                                  sem_ld).wait()
            pltpu.make_async_copy(h_hbm.at[pl.ds(0, UCH)], p1buf[b],
                                  sem_ld).wait()
            pltpu.make_async_copy(dp_hbm.at[0, pl.ds(0, UCH)], d0buf[b],
                                  sem_ld).wait()
            pltpu.make_async_copy(dp_hbm.at[0, pl.ds(0, UCH)], d1buf[b],
                                  sem_ld).wait()

        load_chunk(0, 0)
        for j in range(NJ):
            b = j % 2
            r0 = base + j * UCH
            wait_chunk(b)
            if j + 1 < NJ:
                if j >= 1:
                    pltpu.make_async_copy(hbuf[1 - b],
                                          hout_hbm.at[pl.ds(0, UCH)],
                                          sem_st).wait()
                load_chunk(j + 1, 1 - b)

            def drow(i, _):
                d = d0buf[b][i, :] + d1buf[b][i, :]
                rdbuf[i, :] = one / jnp.maximum(d, one)
                return _

            lax.fori_loop(0, UCH, drow, None)
            _update_rows(hbuf[b], p0buf[b], p1buf[b], rdbuf, UCH)
            pltpu.async_copy(hbuf[b], hout_hbm.at[pl.ds(r0, UCH)], sem_st)
            pltpu.sync_copy(rdbuf, rd_hbm.at[pl.ds(r0, UCH)])
        for b in range(2):
            pltpu.make_async_copy(hbuf[b], hout_hbm.at[pl.ds(0, UCH)],
                                  sem_st).wait()

    return pl.kernel(
        body,
        out_type=(
            jax.ShapeDtypeStruct((NP, D), jnp.float32),
            jax.ShapeDtypeStruct((NP, L), jnp.float32),
        ),
        mesh=_mesh(),
        scratch_types=scratch,
        compiler_params=_params,
    )


def _make_update():
    scratch = [
        [pltpu.VMEM((UCH, D), jnp.float32) for _ in range(2)],
        [pltpu.VMEM((UCH, D), jnp.float32) for _ in range(2)],
        [pltpu.VMEM((UCH, D), jnp.float32) for _ in range(2)],
        [pltpu.VMEM((UCH, L), jnp.float32) for _ in range(2)],
        pltpu.SemaphoreType.DMA,
        pltpu.SemaphoreType.DMA,
    ]

    def body(h_hbm, p_hbm, rd_hbm, hout_hbm, hbuf, p0buf, p1buf, rdbuf,
             sem_ld, sem_st):
        wid = _worker_id()
        base = wid * RPW
        NJ = RPW // UCH

        def load_chunk(j, b):
            r0 = base + j * UCH
            pltpu.async_copy(h_hbm.at[pl.ds(r0, UCH)], hbuf[b], sem_ld)
            pltpu.async_copy(p_hbm.at[0, pl.ds(r0, UCH)], p0buf[b], sem_ld)
            pltpu.async_copy(p_hbm.at[1, pl.ds(r0, UCH)], p1buf[b], sem_ld)
            pltpu.async_copy(rd_hbm.at[pl.ds(r0, UCH)], rdbuf[b], sem_ld)

        def wait_chunk(b):
            pltpu.make_async_copy(h_hbm.at[pl.ds(0, UCH)], hbuf[b],
                                  sem_ld).wait()
            pltpu.make_async_copy(h_hbm.at[pl.ds(0, UCH)], p0buf[b],
                                  sem_ld).wait()
            pltpu.make_async_copy(h_hbm.at[pl.ds(0, UCH)], p1buf[b],
                                  sem_ld).wait()
            pltpu.make_async_copy(rd_hbm.at[pl.ds(0, UCH)], rdbuf[b],
                                  sem_ld).wait()

        load_chunk(0, 0)
        for j in range(NJ):
            b = j % 2
            r0 = base + j * UCH
            wait_chunk(b)
            if j + 1 < NJ:
                if j >= 1:
                    pltpu.make_async_copy(hbuf[1 - b],
                                          hout_hbm.at[pl.ds(0, UCH)],
                                          sem_st).wait()
                load_chunk(j + 1, 1 - b)
            _update_rows(hbuf[b], p0buf[b], p1buf[b], rdbuf[b], UCH)
            pltpu.async_copy(hbuf[b], hout_hbm.at[pl.ds(r0, UCH)], sem_st)
        for b in range(2):
            pltpu.make_async_copy(hbuf[b], hout_hbm.at[pl.ds(0, UCH)],
                                  sem_st).wait()

    return pl.kernel(
        body,
        out_type=jax.ShapeDtypeStruct((NP, D), jnp.float32),
        mesh=_mesh(),
        scratch_types=scratch,
        compiler_params=_params,
    )


_deg = _make_deg()
_scatter = _make_scatter()
_update_first = _make_update_first()
_update = _make_update()


@jax.jit
def kernel(x, edge_index):
    src = edge_index[0]
    dst = edge_index[1]
    xp = jnp.zeros((NP, D), jnp.float32).at[:N].set(x[:, :D])

    zblk = jnp.zeros((RPS, D), jnp.float32)
    dp = _deg(dst)
    p = _scatter(xp, src, dst, zblk)
    h1, rdeg = _update_first(xp, p, dp)
    p2 = _scatter(h1, src, dst, zblk)
    h2 = _update(h1, p2, rdeg)
    p3 = _scatter(h2, src, dst, zblk)
    h3 = _update(h2, p3, rdeg)

    return jnp.concatenate([xp[:N], h1[:N], h2[:N], h3[:N]], axis=1)


# speedup vs baseline: 10.0814x; 1.0230x over previous
import functools

import jax
import jax.numpy as jnp
from jax import lax
from jax.experimental import pallas as pl
from jax.experimental.pallas import tpu as pltpu
from jax.experimental.pallas import tpu_sc as plsc

N = 10000
E = 320000
D = 128
NUM_WL = 3

NC = 2
NS = 16
L = 16
NW = NC * NS

NP = 10240
RPW = NP // NW
UCH = 80
EC = 128
NCHUNK = E // EC
FULL_ROUNDS = NCHUNK // NW
REM = NCHUNK - FULL_ROUNDS * NW
ZR = 128
RPS = NP // NS
ZCOPIES = RPS // ZR

_params = pltpu.CompilerParams(use_tc_tiling_on_sc=False)

_mesh = functools.partial(
    plsc.VectorSubcoreMesh,
    core_axis_name="c",
    subcore_axis_name="s",
    num_cores=NC,
    num_subcores=NS,
)


def _worker_id():
    return lax.axis_index("s") * NC + lax.axis_index("c")


def _fill_2d(ref, nrows, ncols, value):
    vec = jnp.full((L,), value, jnp.float32)

    def row(i, _):
        for cb in range(ncols // L):
            ref[i, pl.ds(cb * L, L)] = vec
        return _

    lax.fori_loop(0, nrows, row, None)


EPW = E // NW
FCH = -(-EPW // EC)
PADE = FCH * EC - EPW
EPWP = FCH * EC
NB = 2


def _scatter_body(h_hbm, src_hbm, dst_hbm, z_hbm, p_hbm, acc_sh, sidx_all,
                  didx, rows, sem_i, sem_g, sem_s):
    cid = lax.axis_index("c")
    sid = lax.axis_index("s")
    wid = _worker_id()
    ebase = wid * EPW

    pltpu.sync_copy(z_hbm, acc_sh.at[pl.ds(sid * RPS, RPS)])
    plsc.subcore_barrier()

    pltpu.sync_copy(src_hbm.at[pl.ds(ebase, EPW)],
                    sidx_all.at[pl.ds(0, EPW)])
    lanes = lax.iota(jnp.int32, L)
    for k in range(PADE // L):
        sidx_all[pl.ds(EPW + k * L, L)] = lanes + (k * L)

    def start_didx(j, b):
        pltpu.async_copy(dst_hbm.at[pl.ds(ebase + j * EC, EC)], didx[b],
                         sem_i)

    def start_gather(j, b):
        pltpu.async_copy(h_hbm.at[sidx_all.at[pl.ds(j * EC, EC)]], rows[b],
                         sem_g)

    def wait_didx(b):
        pltpu.make_async_copy(dst_hbm.at[pl.ds(0, EC)], didx[b],
                              sem_i).wait()

    def wait_gather(b):
        pltpu.make_async_copy(h_hbm.at[sidx_all.at[pl.ds(0, EC)]], rows[b],
                              sem_g).wait()

    def start_scatter(b):
        pltpu.async_copy(rows[b], acc_sh.at[didx[b]], sem_s, add=True)

    def wait_scatter(b):
        pltpu.make_async_copy(rows[b], acc_sh.at[didx[b]], sem_s).wait()

    def load_didx(j, b):
        @pl.when(j < FCH - 1)
        def _():
            start_didx(j, b)

        @pl.when(j == FCH - 1)
        def _():
            pltpu.async_copy(
                dst_hbm.at[pl.ds(ebase + j * EC, EC - PADE)],
                didx[b].at[pl.ds(0, EC - PADE)], sem_i)

    def wait_didx_tail(j, b):
        @pl.when(j < FCH - 1)
        def _():
            wait_didx(b)

        @pl.when(j == FCH - 1)
        def _():
            pltpu.make_async_copy(dst_hbm.at[pl.ds(0, EC - PADE)],
                                  didx[b].at[pl.ds(0, EC - PADE)],
                                  sem_i).wait()
            for k in range(PADE // L):
                didx[b][pl.ds(EC - PADE + k * L, L)] = (
                    lanes + (N + 64 + k * L))

    load_didx(0, 0)
    start_gather(0, 0)

    def pair(g, _):
        for b in range(NB):
            j = g * NB + b

            @pl.when(j < FCH)
            def _():
                wait_didx_tail(j, b)
                wait_gather(b)

            @pl.when(jnp.logical_and(j >= 1, j <= FCH))
            def _():
                wait_scatter(1 - b)

            @pl.when(j < FCH)
            def _():
                start_scatter(b)

            @pl.when(j + 1 < FCH)
            def _():
                load_didx(j + 1, 1 - b)
                start_gather(j + 1, 1 - b)
        return _

    lax.fori_loop(0, (FCH + 1) // NB, pair, None)

    plsc.subcore_barrier()

    r0 = sid * RPS
    pltpu.sync_copy(acc_sh.at[pl.ds(r0, RPS)], p_hbm.at[cid, pl.ds(r0, RPS)])


def _make_scatter():
    out_type = jax.ShapeDtypeStruct((NC, NP, D), jnp.float32)
    scratch = [
        pltpu.VMEM_SHARED((NP, D), jnp.float32),
        pltpu.VMEM((EPWP,), jnp.int32),
        [pltpu.VMEM((EC,), jnp.int32) for _ in range(NB)],
        [pltpu.VMEM((EC, D), jnp.float32) for _ in range(NB)],
        pltpu.SemaphoreType.DMA,
        pltpu.SemaphoreType.DMA,
        pltpu.SemaphoreType.DMA,
    ]

    def body(h, src, dst, z, p, acc, sidx_all, didx, rows, sem_i,
             sem_g, sem_s):
        _scatter_body(h, src, dst, z, p, acc, sidx_all, didx, rows,
                      sem_i, sem_g, sem_s)

    return pl.kernel(
        body,
        out_type=out_type,
        mesh=_mesh(),
        scratch_types=scratch,
        compiler_params=_params,
    )


def _make_deg():
    out_type = jax.ShapeDtypeStruct((NC, NP, L), jnp.float32)
    scratch = [
        pltpu.VMEM_SHARED((NP, L), jnp.float32),
        pltpu.VMEM((RPS, L), jnp.float32),
        [pltpu.VMEM((EC,), jnp.int32) for _ in range(NB)],
        pltpu.VMEM((EC, L), jnp.float32),
        pltpu.SemaphoreType.DMA,
        pltpu.SemaphoreType.DMA,
    ]

    def body(dst_hbm, dp_hbm, dacc_sh, dbuf, didx, ones_b, sem_i, sem_s):
        cid = lax.axis_index("c")
        sid = lax.axis_index("s")
        wid = _worker_id()
        ebase = wid * EPW
        lanes = lax.iota(jnp.int32, L)

        _fill_2d(dbuf, RPS, L, 0.0)
        pltpu.sync_copy(dbuf, dacc_sh.at[pl.ds(sid * RPS, RPS)])
        _fill_2d(ones_b, EC, L, 1.0)
        plsc.subcore_barrier()

        def start_didx(j, b):
            pltpu.async_copy(dst_hbm.at[pl.ds(ebase + j * EC, EC)],
                             didx[b], sem_i)

        def load_didx(j, b):
            @pl.when(j < FCH - 1)
            def _():
                start_didx(j, b)

            @pl.when(j == FCH - 1)
            def _():
                pltpu.async_copy(
                    dst_hbm.at[pl.ds(ebase + j * EC, EC - PADE)],
                    didx[b].at[pl.ds(0, EC - PADE)], sem_i)

        def wait_didx_tail(j, b):
            @pl.when(j < FCH - 1)
            def _():
                pltpu.make_async_copy(dst_hbm.at[pl.ds(0, EC)], didx[b],
                                      sem_i).wait()

            @pl.when(j == FCH - 1)
            def _():
                pltpu.make_async_copy(dst_hbm.at[pl.ds(0, EC - PADE)],
                                      didx[b].at[pl.ds(0, EC - PADE)],
                                      sem_i).wait()
                for k in range(PADE // L):
                    didx[b][pl.ds(EC - PADE + k * L, L)] = (
                        lanes + (N + 64 + k * L))

        def start_scatter(b):
            pltpu.async_copy(ones_b, dacc_sh.at[didx[b]], sem_s, add=True)

        def wait_scatter(b):
            pltpu.make_async_copy(ones_b, dacc_sh.at[didx[b]],
                                  sem_s).wait()

        load_didx(0, 0)

        def pair(g, _):
            for b in range(NB):
                j = g * NB + b

                @pl.when(j < FCH)
                def _():
                    wait_didx_tail(j, b)

                @pl.when(jnp.logical_and(j >= 1, j <= FCH))
                def _():
                    wait_scatter(1 - b)

                @pl.when(j < FCH)
                def _():
                    start_scatter(b)

                @pl.when(j + 1 < FCH)
                def _():
                    load_didx(j + 1, 1 - b)
            return _

        lax.fori_loop(0, (FCH + 1) // NB, pair, None)

        plsc.subcore_barrier()
        pltpu.sync_copy(dacc_sh.at[pl.ds(sid * RPS, RPS)], dbuf)
        pltpu.sync_copy(dbuf, dp_hbm.at[cid, pl.ds(sid * RPS, RPS)])

    return pl.kernel(
        body,
        out_type=out_type,
        mesh=_mesh(),
        scratch_types=scratch,
        compiler_params=_params,
    )


def _update_rows(hbuf, p0buf, p1buf, rdbuf, nrows):
    def row(i, _):
        vd = rdbuf[i, :]
        for cb in range(D // L):
            s = pl.ds(cb * L, L)
            agg = p0buf[i, s] + p1buf[i, s]
            hbuf[i, s] = 0.5 * (hbuf[i, s] + agg * vd)
        return _

    lax.fori_loop(0, nrows, row, None)


def _make_update_first():
    scratch = [
        [pltpu.VMEM((UCH, D), jnp.float32) for _ in range(2)],
        [pltpu.VMEM((UCH, D), jnp.float32) for _ in range(2)],
        [pltpu.VMEM((UCH, D), jnp.float32) for _ in range(2)],
        [pltpu.VMEM((UCH, L), jnp.float32) for _ in range(2)],
        [pltpu.VMEM((UCH, L), jnp.float32) for _ in range(2)],
        pltpu.VMEM((UCH, L), jnp.float32),
        pltpu.SemaphoreType.DMA,
        pltpu.SemaphoreType.DMA,
    ]

    def body(h_hbm, p_hbm, dp_hbm, hout_hbm, rd_hbm, hbuf, p0buf, p1buf,
             d0buf, d1buf, rdbuf, sem_ld, sem_st):
        wid = _worker_id()
        base = wid * RPW
        one = jnp.full((L,), 1.0, jnp.float32)
        NJ = RPW // UCH

        def load_chunk(j, b):
            r0 = base + j * UCH
            pltpu.async_copy(h_hbm.at[pl.ds(r0, UCH)], hbuf[b], sem_ld)
            pltpu.async_copy(p_hbm.at[0, pl.ds(r0, UCH)], p0buf[b], sem_ld)
            pltpu.async_copy(p_hbm.at[1, pl.ds(r0, UCH)], p1buf[b], sem_ld)
            pltpu.async_copy(dp_hbm.at[0, pl.ds(r0, UCH)], d0buf[b],
                             sem_ld)
            pltpu.async_copy(dp_hbm.at[1, pl.ds(r0, UCH)], d1buf[b],
                             sem_ld)

        def wait_chunk(b):
            pltpu.make_async_copy(h_hbm.at[pl.ds(0, UCH)], hbuf[b],
                                  sem_ld).wait()
            pltpu.make_async_copy(h_hbm.at[pl.ds(0, UCH)], p0buf[b],
                                  sem_ld).wait()
            pltpu.make_async_copy(h_hbm.at[pl.ds(0, UCH)], p1buf[b],
                                  sem_ld).wait()
            pltpu.make_async_copy(dp_hbm.at[0, pl.ds(0, UCH)], d0buf[b],
                                  sem_ld).wait()
            pltpu.make_async_copy(dp_hbm.at[0, pl.ds(0, UCH)], d1buf[b],
                                  sem_ld).wait()

        load_chunk(0, 0)
        for j in range(NJ):
            b = j % 2
            r0 = base + j * UCH
            wait_chunk(b)
            if j + 1 < NJ:
                if j >= 1:
                    pltpu.make_async_copy(hbuf[1 - b],
                                          hout_hbm.at[pl.ds(0, UCH)],
                                          sem_st).wait()
                load_chunk(j + 1, 1 - b)

            def drow(i, _):
                d = d0buf[b][i, :] + d1buf[b][i, :]
                rdbuf[i, :] = one / jnp.maximum(d, one)
                return _

            lax.fori_loop(0, UCH, drow, None)
            _update_rows(hbuf[b], p0buf[b], p1buf[b], rdbuf, UCH)
            pltpu.async_copy(hbuf[b], hout_hbm.at[pl.ds(r0, UCH)], sem_st)
            pltpu.sync_copy(rdbuf, rd_hbm.at[pl.ds(r0, UCH)])
        for b in range(2):
            pltpu.make_async_copy(hbuf[b], hout_hbm.at[pl.ds(0, UCH)],
                                  sem_st).wait()

    return pl.kernel(
        body,
        out_type=(
            jax.ShapeDtypeStruct((NP, D), jnp.float32),
            jax.ShapeDtypeStruct((NP, L), jnp.float32),
        ),
        mesh=_mesh(),
        scratch_types=scratch,
        compiler_params=_params,
    )


def _make_update():
    scratch = [
        [pltpu.VMEM((UCH, D), jnp.float32) for _ in range(2)],
        [pltpu.VMEM((UCH, D), jnp.float32) for _ in range(2)],
        [pltpu.VMEM((UCH, D), jnp.float32) for _ in range(2)],
        [pltpu.VMEM((UCH, L), jnp.float32) for _ in range(2)],
        pltpu.SemaphoreType.DMA,
        pltpu.SemaphoreType.DMA,
    ]

    def body(h_hbm, p_hbm, rd_hbm, hout_hbm, hbuf, p0buf, p1buf, rdbuf,
             sem_ld, sem_st):
        wid = _worker_id()
        base = wid * RPW
        NJ = RPW // UCH

        def load_chunk(j, b):
            r0 = base + j * UCH
            pltpu.async_copy(h_hbm.at[pl.ds(r0, UCH)], hbuf[b], sem_ld)
            pltpu.async_copy(p_hbm.at[0, pl.ds(r0, UCH)], p0buf[b], sem_ld)
            pltpu.async_copy(p_hbm.at[1, pl.ds(r0, UCH)], p1buf[b], sem_ld)
            pltpu.async_copy(rd_hbm.at[pl.ds(r0, UCH)], rdbuf[b], sem_ld)

        def wait_chunk(b):
            pltpu.make_async_copy(h_hbm.at[pl.ds(0, UCH)], hbuf[b],
                                  sem_ld).wait()
            pltpu.make_async_copy(h_hbm.at[pl.ds(0, UCH)], p0buf[b],
                                  sem_ld).wait()
            pltpu.make_async_copy(h_hbm.at[pl.ds(0, UCH)], p1buf[b],
                                  sem_ld).wait()
            pltpu.make_async_copy(rd_hbm.at[pl.ds(0, UCH)], rdbuf[b],
                                  sem_ld).wait()

        load_chunk(0, 0)
        for j in range(NJ):
            b = j % 2
            r0 = base + j * UCH
            wait_chunk(b)
            if j + 1 < NJ:
                if j >= 1:
                    pltpu.make_async_copy(hbuf[1 - b],
                                          hout_hbm.at[pl.ds(0, UCH)],
                                          sem_st).wait()
                load_chunk(j + 1, 1 - b)
            _update_rows(hbuf[b], p0buf[b], p1buf[b], rdbuf[b], UCH)
            pltpu.async_copy(hbuf[b], hout_hbm.at[pl.ds(r0, UCH)], sem_st)
        for b in range(2):
            pltpu.make_async_copy(hbuf[b], hout_hbm.at[pl.ds(0, UCH)],
                                  sem_st).wait()

    return pl.kernel(
        body,
        out_type=jax.ShapeDtypeStruct((NP, D), jnp.float32),
        mesh=_mesh(),
        scratch_types=scratch,
        compiler_params=_params,
    )


_deg = _make_deg()
_scatter = _make_scatter()
_update_first = _make_update_first()
_update = _make_update()


@jax.jit
def kernel(x, edge_index):
    src = edge_index[0]
    dst = edge_index[1]
    xp = jnp.zeros((NP, D), jnp.float32).at[:N].set(x[:, :D])

    zblk = jnp.zeros((RPS, D), jnp.float32)
    dp = _deg(dst)
    p = _scatter(xp, src, dst, zblk)
    h1, rdeg = _update_first(xp, p, dp)
    p2 = _scatter(h1, src, dst, zblk)
    h2 = _update(h1, p2, rdeg)
    p3 = _scatter(h2, src, dst, zblk)
    h3 = _update(h2, p3, rdeg)

    return jnp.concatenate([xp[:N], h1[:N], h2[:N], h3[:N]], axis=1)


# speedup vs baseline: 10.0924x; 1.0011x over previous
import functools

import jax
import jax.numpy as jnp
from jax import lax
from jax.experimental import pallas as pl
from jax.experimental.pallas import tpu as pltpu
from jax.experimental.pallas import tpu_sc as plsc

N = 10000
E = 320000
D = 128
NUM_WL = 3

NC = 2
NS = 16
L = 16
NW = NC * NS

NP = 10240
RPW = NP // NW
UCH = 80
EC = 128
NCHUNK = E // EC
FULL_ROUNDS = NCHUNK // NW
REM = NCHUNK - FULL_ROUNDS * NW
ZR = 128
RPS = NP // NS
ZCOPIES = RPS // ZR

_params = pltpu.CompilerParams(use_tc_tiling_on_sc=False)

_mesh = functools.partial(
    plsc.VectorSubcoreMesh,
    core_axis_name="c",
    subcore_axis_name="s",
    num_cores=NC,
    num_subcores=NS,
)


def _worker_id():
    return lax.axis_index("s") * NC + lax.axis_index("c")


def _fill_2d(ref, nrows, ncols, value):
    vec = jnp.full((L,), value, jnp.float32)

    def row(i, _):
        for cb in range(ncols // L):
            ref[i, pl.ds(cb * L, L)] = vec
        return _

    lax.fori_loop(0, nrows, row, None)


EPW = E // NW
FCH = -(-EPW // EC)
PADE = FCH * EC - EPW
EPWP = FCH * EC
NB = 2


def _scatter_body(h_hbm, src_hbm, dst_hbm, z_hbm, p_hbm, acc_sh, sidx_all,
                  didx, rows, sem_i, sem_g, sem_s):
    cid = lax.axis_index("c")
    sid = lax.axis_index("s")
    wid = _worker_id()
    ebase = wid * EPW

    pltpu.sync_copy(z_hbm, acc_sh.at[pl.ds(sid * RPS, RPS)])
    plsc.subcore_barrier()

    pltpu.sync_copy(src_hbm.at[pl.ds(ebase, EPW)],
                    sidx_all.at[pl.ds(0, EPW)])
    lanes = lax.iota(jnp.int32, L)
    for k in range(PADE // L):
        sidx_all[pl.ds(EPW + k * L, L)] = lanes + (k * L)

    def start_didx(j, b):
        pltpu.async_copy(dst_hbm.at[pl.ds(ebase + j * EC, EC)], didx[b],
                         sem_i)

    def start_gather(j, b):
        pltpu.async_copy(h_hbm.at[sidx_all.at[pl.ds(j * EC, EC)]], rows[b],
                         sem_g)

    def wait_didx(b):
        pltpu.make_async_copy(dst_hbm.at[pl.ds(0, EC)], didx[b],
                              sem_i).wait()

    def wait_gather(b):
        pltpu.make_async_copy(h_hbm.at[sidx_all.at[pl.ds(0, EC)]], rows[b],
                              sem_g).wait()

    def start_scatter(b):
        pltpu.async_copy(rows[b], acc_sh.at[didx[b]], sem_s, add=True)

    def wait_scatter(b):
        pltpu.make_async_copy(rows[b], acc_sh.at[didx[b]], sem_s).wait()

    def load_didx(j, b):
        @pl.when(j < FCH - 1)
        def _():
            start_didx(j, b)

        @pl.when(j == FCH - 1)
        def _():
            pltpu.async_copy(
                dst_hbm.at[pl.ds(ebase + j * EC, EC - PADE)],
                didx[b].at[pl.ds(0, EC - PADE)], sem_i)

    def wait_didx_tail(j, b):
        @pl.when(j < FCH - 1)
        def _():
            wait_didx(b)

        @pl.when(j == FCH - 1)
        def _():
            pltpu.make_async_copy(dst_hbm.at[pl.ds(0, EC - PADE)],
                                  didx[b].at[pl.ds(0, EC - PADE)],
                                  sem_i).wait()
            for k in range(PADE // L):
                didx[b][pl.ds(EC - PADE + k * L, L)] = (
                    lanes + (N + 64 + k * L))

    load_didx(0, 0)
    start_gather(0, 0)

    def pair(g, _):
        for b in range(NB):
            j = g * NB + b

            @pl.when(j < FCH)
            def _():
                wait_didx_tail(j, b)
                wait_gather(b)

            @pl.when(jnp.logical_and(j >= 1, j <= FCH))
            def _():
                wait_scatter(1 - b)

            @pl.when(j + 1 < FCH)
            def _():
                load_didx(j + 1, 1 - b)
                start_gather(j + 1, 1 - b)

            @pl.when(j < FCH)
            def _():
                start_scatter(b)
        return _

    lax.fori_loop(0, (FCH + 1) // NB, pair, None)

    plsc.subcore_barrier()

    r0 = sid * RPS
    pltpu.sync_copy(acc_sh.at[pl.ds(r0, RPS)], p_hbm.at[cid, pl.ds(r0, RPS)])


def _make_scatter():
    out_type = jax.ShapeDtypeStruct((NC, NP, D), jnp.float32)
    scratch = [
        pltpu.VMEM_SHARED((NP, D), jnp.float32),
        pltpu.VMEM((EPWP,), jnp.int32),
        [pltpu.VMEM((EC,), jnp.int32) for _ in range(NB)],
        [pltpu.VMEM((EC, D), jnp.float32) for _ in range(NB)],
        pltpu.SemaphoreType.DMA,
        pltpu.SemaphoreType.DMA,
        pltpu.SemaphoreType.DMA,
    ]

    def body(h, src, dst, z, p, acc, sidx_all, didx, rows, sem_i,
             sem_g, sem_s):
        _scatter_body(h, src, dst, z, p, acc, sidx_all, didx, rows,
                      sem_i, sem_g, sem_s)

    return pl.kernel(
        body,
        out_type=out_type,
        mesh=_mesh(),
        scratch_types=scratch,
        compiler_params=_params,
    )


def _make_deg():
    out_type = jax.ShapeDtypeStruct((NC, NP, L), jnp.float32)
    scratch = [
        pltpu.VMEM_SHARED((NP, L), jnp.float32),
        pltpu.VMEM((RPS, L), jnp.float32),
        [pltpu.VMEM((EC,), jnp.int32) for _ in range(NB)],
        pltpu.VMEM((EC, L), jnp.float32),
        pltpu.SemaphoreType.DMA,
        pltpu.SemaphoreType.DMA,
    ]

    def body(dst_hbm, dp_hbm, dacc_sh, dbuf, didx, ones_b, sem_i, sem_s):
        cid = lax.axis_index("c")
        sid = lax.axis_index("s")
        wid = _worker_id()
        ebase = wid * EPW
        lanes = lax.iota(jnp.int32, L)

        _fill_2d(dbuf, RPS, L, 0.0)
        pltpu.sync_copy(dbuf, dacc_sh.at[pl.ds(sid * RPS, RPS)])
        _fill_2d(ones_b, EC, L, 1.0)
        plsc.subcore_barrier()

        def start_didx(j, b):
            pltpu.async_copy(dst_hbm.at[pl.ds(ebase + j * EC, EC)],
                             didx[b], sem_i)

        def load_didx(j, b):
            @pl.when(j < FCH - 1)
            def _():
                start_didx(j, b)

            @pl.when(j == FCH - 1)
            def _():
                pltpu.async_copy(
                    dst_hbm.at[pl.ds(ebase + j * EC, EC - PADE)],
                    didx[b].at[pl.ds(0, EC - PADE)], sem_i)

        def wait_didx_tail(j, b):
            @pl.when(j < FCH - 1)
            def _():
                pltpu.make_async_copy(dst_hbm.at[pl.ds(0, EC)], didx[b],
                                      sem_i).wait()

            @pl.when(j == FCH - 1)
            def _():
                pltpu.make_async_copy(dst_hbm.at[pl.ds(0, EC - PADE)],
                                      didx[b].at[pl.ds(0, EC - PADE)],
                                      sem_i).wait()
                for k in range(PADE // L):
                    didx[b][pl.ds(EC - PADE + k * L, L)] = (
                        lanes + (N + 64 + k * L))

        def start_scatter(b):
            pltpu.async_copy(ones_b, dacc_sh.at[didx[b]], sem_s, add=True)

        def wait_scatter(b):
            pltpu.make_async_copy(ones_b, dacc_sh.at[didx[b]],
                                  sem_s).wait()

        load_didx(0, 0)

        def pair(g, _):
            for b in range(NB):
                j = g * NB + b

                @pl.when(j < FCH)
                def _():
                    wait_didx_tail(j, b)

                @pl.when(jnp.logical_and(j >= 1, j <= FCH))
                def _():
                    wait_scatter(1 - b)

                @pl.when(j < FCH)
                def _():
                    start_scatter(b)

                @pl.when(j + 1 < FCH)
                def _():
                    load_didx(j + 1, 1 - b)
            return _

        lax.fori_loop(0, (FCH + 1) // NB, pair, None)

        plsc.subcore_barrier()
        pltpu.sync_copy(dacc_sh.at[pl.ds(sid * RPS, RPS)], dbuf)
        pltpu.sync_copy(dbuf, dp_hbm.at[cid, pl.ds(sid * RPS, RPS)])

    return pl.kernel(
        body,
        out_type=out_type,
        mesh=_mesh(),
        scratch_types=scratch,
        compiler_params=_params,
    )


def _update_rows(hbuf, p0buf, p1buf, rdbuf, nrows):
    def row(i, _):
        vd = rdbuf[i, :]
        for cb in range(D // L):
            s = pl.ds(cb * L, L)
            agg = p0buf[i, s] + p1buf[i, s]
            hbuf[i, s] = 0.5 * (hbuf[i, s] + agg * vd)
        return _

    lax.fori_loop(0, nrows, row, None)


def _make_update_first():
    scratch = [
        [pltpu.VMEM((UCH, D), jnp.float32) for _ in range(2)],
        [pltpu.VMEM((UCH, D), jnp.float32) for _ in range(2)],
        [pltpu.VMEM((UCH, D), jnp.float32) for _ in range(2)],
        [pltpu.VMEM((UCH, L), jnp.float32) for _ in range(2)],
        [pltpu.VMEM((UCH, L), jnp.float32) for _ in range(2)],
        pltpu.VMEM((UCH, L), jnp.float32),
        pltpu.SemaphoreType.DMA,
        pltpu.SemaphoreType.DMA,
    ]

    def body(h_hbm, p_hbm, dp_hbm, hout_hbm, rd_hbm, hbuf, p0buf, p1buf,
             d0buf, d1buf, rdbuf, sem_ld, sem_st):
        wid = _worker_id()
        base = wid * RPW
        one = jnp.full((L,), 1.0, jnp.float32)
        NJ = RPW // UCH

        def load_chunk(j, b):
            r0 = base + j * UCH
            pltpu.async_copy(h_hbm.at[pl.ds(r0, UCH)], hbuf[b], sem_ld)
            pltpu.async_copy(p_hbm.at[0, pl.ds(r0, UCH)], p0buf[b], sem_ld)
            pltpu.async_copy(p_hbm.at[1, pl.ds(r0, UCH)], p1buf[b], sem_ld)
            pltpu.async_copy(dp_hbm.at[0, pl.ds(r0, UCH)], d0buf[b],
                             sem_ld)
            pltpu.async_copy(dp_hbm.at[1, pl.ds(r0, UCH)], d1buf[b],
                             sem_ld)

        def wait_chunk(b):
            pltpu.make_async_copy(h_hbm.at[pl.ds(0, UCH)], hbuf[b],
                                  sem_ld).wait()
            pltpu.make_async_copy(h_hbm.at[pl.ds(0, UCH)], p0buf[b],
                                  sem_ld).wait()
            pltpu.make_async_copy(h_hbm.at[pl.ds(0, UCH)], p1buf[b],
                                  sem_ld).wait()
            pltpu.make_async_copy(dp_hbm.at[0, pl.ds(0, UCH)], d0buf[b],
                                  sem_ld).wait()
            pltpu.make_async_copy(dp_hbm.at[0, pl.ds(0, UCH)], d1buf[b],
                                  sem_ld).wait()

        load_chunk(0, 0)
        for j in range(NJ):
            b = j % 2
            r0 = base + j * UCH
            wait_chunk(b)
            if j + 1 < NJ:
                if j >= 1:
                    pltpu.make_async_copy(hbuf[1 - b],
                                          hout_hbm.at[pl.ds(0, UCH)],
                                          sem_st).wait()
                load_chunk(j + 1, 1 - b)

            def drow(i, _):
                d = d0buf[b][i, :] + d1buf[b][i, :]
                rdbuf[i, :] = one / jnp.maximum(d, one)
                return _

            lax.fori_loop(0, UCH, drow, None)
            _update_rows(hbuf[b], p0buf[b], p1buf[b], rdbuf, UCH)
            pltpu.async_copy(hbuf[b], hout_hbm.at[pl.ds(r0, UCH)], sem_st)
            pltpu.sync_copy(rdbuf, rd_hbm.at[pl.ds(r0, UCH)])
        for b in range(2):
            pltpu.make_async_copy(hbuf[b], hout_hbm.at[pl.ds(0, UCH)],
                                  sem_st).wait()

    return pl.kernel(
        body,
        out_type=(
            jax.ShapeDtypeStruct((NP, D), jnp.float32),
            jax.ShapeDtypeStruct((NP, L), jnp.float32),
        ),
        mesh=_mesh(),
        scratch_types=scratch,
        compiler_params=_params,
    )


def _make_update():
    scratch = [
        [pltpu.VMEM((UCH, D), jnp.float32) for _ in range(2)],
        [pltpu.VMEM((UCH, D), jnp.float32) for _ in range(2)],
        [pltpu.VMEM((UCH, D), jnp.float32) for _ in range(2)],
        [pltpu.VMEM((UCH, L), jnp.float32) for _ in range(2)],
        pltpu.SemaphoreType.DMA,
        pltpu.SemaphoreType.DMA,
    ]

    def body(h_hbm, p_hbm, rd_hbm, hout_hbm, hbuf, p0buf, p1buf, rdbuf,
             sem_ld, sem_st):
        wid = _worker_id()
        base = wid * RPW
        NJ = RPW // UCH

        def load_chunk(j, b):
            r0 = base + j * UCH
            pltpu.async_copy(h_hbm.at[pl.ds(r0, UCH)], hbuf[b], sem_ld)
            pltpu.async_copy(p_hbm.at[0, pl.ds(r0, UCH)], p0buf[b], sem_ld)
            pltpu.async_copy(p_hbm.at[1, pl.ds(r0, UCH)], p1buf[b], sem_ld)
            pltpu.async_copy(rd_hbm.at[pl.ds(r0, UCH)], rdbuf[b], sem_ld)

        def wait_chunk(b):
            pltpu.make_async_copy(h_hbm.at[pl.ds(0, UCH)], hbuf[b],
                                  sem_ld).wait()
            pltpu.make_async_copy(h_hbm.at[pl.ds(0, UCH)], p0buf[b],
                                  sem_ld).wait()
            pltpu.make_async_copy(h_hbm.at[pl.ds(0, UCH)], p1buf[b],
                                  sem_ld).wait()
            pltpu.make_async_copy(rd_hbm.at[pl.ds(0, UCH)], rdbuf[b],
                                  sem_ld).wait()

        load_chunk(0, 0)
        for j in range(NJ):
            b = j % 2
            r0 = base + j * UCH
            wait_chunk(b)
            if j + 1 < NJ:
                if j >= 1:
                    pltpu.make_async_copy(hbuf[1 - b],
                                          hout_hbm.at[pl.ds(0, UCH)],
                                          sem_st).wait()
                load_chunk(j + 1, 1 - b)
            _update_rows(hbuf[b], p0buf[b], p1buf[b], rdbuf[b], UCH)
            pltpu.async_copy(hbuf[b], hout_hbm.at[pl.ds(r0, UCH)], sem_st)
        for b in range(2):
            pltpu.make_async_copy(hbuf[b], hout_hbm.at[pl.ds(0, UCH)],
                                  sem_st).wait()

    return pl.kernel(
        body,
        out_type=jax.ShapeDtypeStruct((NP, D), jnp.float32),
        mesh=_mesh(),
        scratch_types=scratch,
        compiler_params=_params,
    )


_deg = _make_deg()
_scatter = _make_scatter()
_update_first = _make_update_first()
_update = _make_update()


@jax.jit
def kernel(x, edge_index):
    src = edge_index[0]
    dst = edge_index[1]
    xp = jnp.zeros((NP, D), jnp.float32).at[:N].set(x[:, :D])

    zblk = jnp.zeros((RPS, D), jnp.float32)
    dp = _deg(dst)
    p = _scatter(xp, src, dst, zblk)
    h1, rdeg = _update_first(xp, p, dp)
    p2 = _scatter(h1, src, dst, zblk)
    h2 = _update(h1, p2, rdeg)
    p3 = _scatter(h2, src, dst, zblk)
    h3 = _update(h2, p3, rdeg)

    return jnp.concatenate([xp[:N], h1[:N], h2[:N], h3[:N]], axis=1)
